# baseline (speedup 1.0000x reference)
import functools

import jax
import jax.numpy as jnp
from jax import lax
from jax.experimental import pallas as pl
from jax.experimental.pallas import tpu as pltpu
from jax.experimental.pallas import tpu_sc as plsc

_E = 8
_H = 1024
_F = 1024
_TOPK = 2
_S = 2048
_A = _S * _TOPK

_BLK = 256
_NPAD = 6144
_NBLK = _NPAD // _BLK

_NTILES = 32
_RPT = _NPAD // _NTILES
_GCH = 64
_NGCH = _RPT // _GCH
_TPT = _S // _NTILES
_CCH = 32
_NCCH = _TPT // _CCH
_LANES = 16


def _route(top_k_index, top_k_weights):
    flat_e = top_k_index.reshape(-1).astype(jnp.int32)
    order = jnp.argsort(flat_e).astype(jnp.int32)
    e_s = flat_e[order]
    tok_s = (order // _TOPK).astype(jnp.int32)
    w_s = top_k_weights.reshape(-1)[order]

    cnt = jnp.bincount(flat_e, length=_E).astype(jnp.int32)
    start = jnp.concatenate([jnp.zeros(1, jnp.int32), jnp.cumsum(cnt)[:-1]])
    bcnt = (cnt + _BLK - 1) // _BLK
    bstart = jnp.concatenate([jnp.zeros(1, jnp.int32), jnp.cumsum(bcnt)[:-1]])
    pstart = bstart * _BLK

    pad_pos = (pstart[e_s] + (jnp.arange(_A, dtype=jnp.int32) - start[e_s]))
    row_token = jnp.zeros(_NPAD, jnp.int32).at[pad_pos].set(tok_s)
    row_weight = jnp.zeros(_NPAD, jnp.float32).at[pad_pos].set(w_s)
    pos_orig = jnp.zeros(_A, jnp.int32).at[order].set(pad_pos)

    used = jnp.sum(bcnt)
    bids = jnp.arange(_NBLK, dtype=jnp.int32)
    be_raw = jnp.searchsorted(jnp.cumsum(bcnt), bids, side="right").astype(jnp.int32)
    be_last = jnp.take(be_raw, used - 1)
    be = jnp.where(bids < used, be_raw, be_last)
    xi = jnp.where(bids < used, bids, used - 1).astype(jnp.int32)
    vld = (bids < used).astype(jnp.int32)
    meta = jnp.stack([be, xi, vld])
    return row_token, row_weight, pos_orig, meta


@functools.cache
def _gather_rows_kernel():
    mesh = plsc.VectorSubcoreMesh(core_axis_name="c", subcore_axis_name="s")

    @functools.partial(
        pl.kernel,
        mesh=mesh,
        out_type=jax.ShapeDtypeStruct((_NPAD, _F), jnp.float32),
        scratch_types=[
            pltpu.VMEM((_NGCH, _GCH), jnp.int32),
            pltpu.VMEM((_GCH, _F), jnp.float32),
            pltpu.SemaphoreType.DMA,
        ],
    )
    def _gather_rows(tok_hbm, hs_hbm, x_hbm, idx_v, buf_v, sem):
        wid = lax.axis_index("s") * 2 + lax.axis_index("c")
        pltpu.sync_copy(tok_hbm.at[wid], idx_v)
        base = wid * _RPT
        for c in range(_NGCH):
            pltpu.async_copy(hs_hbm.at[idx_v.at[c]], buf_v, sem).wait()
            pltpu.sync_copy(buf_v, x_hbm.at[pl.ds(base + c * _GCH, _GCH)])

    return _gather_rows


def _expert_body(meta_ref, x_ref, w1_ref, v1_ref, w2_ref, rw_ref, y_ref):
    i = pl.program_id(0)

    @pl.when(meta_ref[2, i] == 1)
    def _():
        x = x_ref[...]
        g = jnp.dot(x, w1_ref[0], preferred_element_type=jnp.float32)
        u = jnp.dot(x, v1_ref[0], preferred_element_type=jnp.float32)
        inter = (g * jax.nn.sigmoid(g)) * u
        y = lax.dot_general(inter, w2_ref[0], (((1,), (1,)), ((), ())),
                            preferred_element_type=jnp.float32)
        y_ref[...] = y * rw_ref[...]


def _expert_matmul(meta, x, w1r, v1r, w2r, rw):
    grid_spec = pltpu.PrefetchScalarGridSpec(
        num_scalar_prefetch=1,
        grid=(_NBLK,),
        in_specs=[
            pl.BlockSpec((_BLK, _F), lambda i, m: (m[1, i], 0)),
            pl.BlockSpec((1, _F, _H), lambda i, m: (m[0, i], 0, 0)),
            pl.BlockSpec((1, _F, _H), lambda i, m: (m[0, i], 0, 0)),
            pl.BlockSpec((1, _F, _H), lambda i, m: (m[0, i], 0, 0)),
            pl.BlockSpec((_BLK, 1), lambda i, m: (m[1, i], 0)),
        ],
        out_specs=pl.BlockSpec((_BLK, _F), lambda i, m: (i, 0)),
    )
    return pl.pallas_call(
        _expert_body,
        grid_spec=grid_spec,
        out_shape=jax.ShapeDtypeStruct((_NPAD, _F), jnp.float32),
    )(meta, x, w1r, v1r, w2r, rw)


@functools.cache
def _combine_kernel():
    mesh = plsc.VectorSubcoreMesh(core_axis_name="c", subcore_axis_name="s")

    @functools.partial(
        pl.kernel,
        mesh=mesh,
        out_type=jax.ShapeDtypeStruct((_S, _F), jnp.float32),
        scratch_types=[
            pltpu.VMEM((_NCCH, _CCH), jnp.int32),
            pltpu.VMEM((_NCCH, _CCH), jnp.int32),
            pltpu.VMEM((_CCH, _F), jnp.float32),
            pltpu.VMEM((_CCH, _F), jnp.float32),
            pltpu.SemaphoreType.DMA,
            pltpu.SemaphoreType.DMA,
        ],
    )
    def _combine(posa_hbm, posb_hbm, y_hbm, out_hbm, ia_v, ib_v, ba_v, bb_v, sa, sb):
        wid = lax.axis_index("s") * 2 + lax.axis_index("c")
        pltpu.sync_copy(posa_hbm.at[wid], ia_v)
        pltpu.sync_copy(posb_hbm.at[wid], ib_v)
        base = wid * _TPT
        for c in range(_NCCH):
            cpa = pltpu.async_copy(y_hbm.at[ia_v.at[c]], ba_v, sa)
            cpb = pltpu.async_copy(y_hbm.at[ib_v.at[c]], bb_v, sb)
            cpa.wait()
            cpb.wait()

            def row_body(r, carry):
                for d in range(_F // _LANES):
                    sl = pl.ds(d * _LANES, _LANES)
                    ba_v[r, sl] = ba_v[r, sl] + bb_v[r, sl]
                return carry

            lax.fori_loop(0, _CCH, row_body, 0)
            pltpu.sync_copy(ba_v, out_hbm.at[pl.ds(base + c * _CCH, _CCH)])

    return _combine


def kernel(hidden_states, top_k_index, top_k_weights, w1, v1, w2):
    bsz = hidden_states.shape[0]
    hs = hidden_states.reshape(_S, _F)
    row_token, row_weight, pos_orig, meta = _route(top_k_index, top_k_weights)

    x = _gather_rows_kernel()(row_token.reshape(_NTILES, _NGCH, _GCH), hs)

    w1r = w1.reshape(_E, _F, _H)
    v1r = v1.reshape(_E, _F, _H)
    w2r = w2.reshape(_E, _F, _H)
    y = _expert_matmul(meta, x, w1r, v1r, w2r, row_weight.reshape(_NPAD, 1))

    pos2 = pos_orig.reshape(_S, _TOPK)
    posa = pos2[:, 0].reshape(_NTILES, _NCCH, _CCH)
    posb = pos2[:, 1].reshape(_NTILES, _NCCH, _CCH)
    out = _combine_kernel()(posa, posb, y)
    return out.reshape(bsz, _S, _F)


# speedup vs baseline: 1.0062x; 1.0062x over previous
import functools

import jax
import jax.numpy as jnp
from jax import lax
from jax.experimental import pallas as pl
from jax.experimental.pallas import tpu as pltpu
from jax.experimental.pallas import tpu_sc as plsc

_E = 8
_H = 1024
_F = 1024
_TOPK = 2
_S = 2048
_A = _S * _TOPK

_BLK = 256
_NPAD = 6144
_NBLK = _NPAD // _BLK

_NTILES = 32
_RPT = _NPAD // _NTILES
_GCH = 48
_NGCH = _RPT // _GCH
_TPT = _S // _NTILES
_CCH = 16
_NCCH = _TPT // _CCH
_LANES = 16


def _route(top_k_index, top_k_weights):
    flat_e = top_k_index.reshape(-1).astype(jnp.int32)
    order = jnp.argsort(flat_e).astype(jnp.int32)
    e_s = flat_e[order]
    tok_s = (order // _TOPK).astype(jnp.int32)
    w_s = top_k_weights.reshape(-1)[order]

    cnt = jnp.bincount(flat_e, length=_E).astype(jnp.int32)
    start = jnp.concatenate([jnp.zeros(1, jnp.int32), jnp.cumsum(cnt)[:-1]])
    bcnt = (cnt + _BLK - 1) // _BLK
    bstart = jnp.concatenate([jnp.zeros(1, jnp.int32), jnp.cumsum(bcnt)[:-1]])
    pstart = bstart * _BLK

    pad_pos = (pstart[e_s] + (jnp.arange(_A, dtype=jnp.int32) - start[e_s]))
    row_token = jnp.zeros(_NPAD, jnp.int32).at[pad_pos].set(tok_s)
    row_weight = jnp.zeros(_NPAD, jnp.float32).at[pad_pos].set(w_s)
    pos_orig = jnp.zeros(_A, jnp.int32).at[order].set(pad_pos)

    used = jnp.sum(bcnt)
    bids = jnp.arange(_NBLK, dtype=jnp.int32)
    be_raw = jnp.searchsorted(jnp.cumsum(bcnt), bids, side="right").astype(jnp.int32)
    be_last = jnp.take(be_raw, used - 1)
    be = jnp.where(bids < used, be_raw, be_last)
    xi = jnp.where(bids < used, bids, used - 1).astype(jnp.int32)
    vld = (bids < used).astype(jnp.int32)
    meta = jnp.stack([be, xi, vld])
    return row_token, row_weight, pos_orig, meta


@functools.cache
def _gather_rows_kernel():
    mesh = plsc.VectorSubcoreMesh(core_axis_name="c", subcore_axis_name="s")

    @functools.partial(
        pl.kernel,
        mesh=mesh,
        out_type=jax.ShapeDtypeStruct((_NPAD, _F), jnp.float32),
        scratch_types=[
            pltpu.VMEM((_NGCH, _GCH), jnp.int32),
            pltpu.VMEM((_GCH, _F), jnp.float32),
            pltpu.VMEM((_GCH, _F), jnp.float32),
            pltpu.SemaphoreType.DMA,
            pltpu.SemaphoreType.DMA,
            pltpu.SemaphoreType.DMA,
            pltpu.SemaphoreType.DMA,
        ],
    )
    def _gather_rows(tok_hbm, hs_hbm, x_hbm, idx_v, b0, b1, sg0, sg1, sw0, sw1):
        wid = lax.axis_index("s") * 2 + lax.axis_index("c")
        pltpu.sync_copy(tok_hbm.at[wid], idx_v)
        base = wid * _RPT
        bufs = (b0, b1)
        sgs = (sg0, sg1)
        sws = (sw0, sw1)
        gat = [None, None]
        wrt = [None, None]
        for c in range(_NGCH):
            p = c % 2
            if wrt[p] is not None:
                wrt[p].wait()
            gat[p] = pltpu.async_copy(hs_hbm.at[idx_v.at[c]], bufs[p], sgs[p])
            if c >= 1:
                q = (c - 1) % 2
                gat[q].wait()
                wrt[q] = pltpu.async_copy(
                    bufs[q], x_hbm.at[pl.ds(base + (c - 1) * _GCH, _GCH)], sws[q])
        last = (_NGCH - 1) % 2
        gat[last].wait()
        wrt[last] = pltpu.async_copy(
            bufs[last], x_hbm.at[pl.ds(base + (_NGCH - 1) * _GCH, _GCH)], sws[last])
        wrt[0].wait()
        wrt[1].wait()

    return _gather_rows


def _expert_body(meta_ref, x_ref, w1_ref, v1_ref, w2_ref, rw_ref, y_ref):
    i = pl.program_id(0)

    @pl.when(meta_ref[2, i] == 1)
    def _():
        x = x_ref[...]
        g = jnp.dot(x, w1_ref[0], preferred_element_type=jnp.float32)
        u = jnp.dot(x, v1_ref[0], preferred_element_type=jnp.float32)
        inter = (g * jax.nn.sigmoid(g)) * u
        y = lax.dot_general(inter, w2_ref[0], (((1,), (1,)), ((), ())),
                            preferred_element_type=jnp.float32)
        y_ref[...] = y * rw_ref[...]


def _expert_matmul(meta, x, w1r, v1r, w2r, rw):
    grid_spec = pltpu.PrefetchScalarGridSpec(
        num_scalar_prefetch=1,
        grid=(_NBLK,),
        in_specs=[
            pl.BlockSpec((_BLK, _F), lambda i, m: (m[1, i], 0)),
            pl.BlockSpec((1, _F, _H), lambda i, m: (m[0, i], 0, 0)),
            pl.BlockSpec((1, _F, _H), lambda i, m: (m[0, i], 0, 0)),
            pl.BlockSpec((1, _F, _H), lambda i, m: (m[0, i], 0, 0)),
            pl.BlockSpec((_BLK, 1), lambda i, m: (m[1, i], 0)),
        ],
        out_specs=pl.BlockSpec((_BLK, _F), lambda i, m: (i, 0)),
    )
    return pl.pallas_call(
        _expert_body,
        grid_spec=grid_spec,
        out_shape=jax.ShapeDtypeStruct((_NPAD, _F), jnp.float32),
    )(meta, x, w1r, v1r, w2r, rw)


@functools.cache
def _combine_kernel():
    mesh = plsc.VectorSubcoreMesh(core_axis_name="c", subcore_axis_name="s")

    @functools.partial(
        pl.kernel,
        mesh=mesh,
        out_type=jax.ShapeDtypeStruct((_S, _F), jnp.float32),
        scratch_types=[
            pltpu.VMEM((_NCCH, _CCH), jnp.int32),
            pltpu.VMEM((_NCCH, _CCH), jnp.int32),
            pltpu.VMEM((_CCH, _F), jnp.float32),
            pltpu.VMEM((_CCH, _F), jnp.float32),
            pltpu.VMEM((_CCH, _F), jnp.float32),
            pltpu.VMEM((_CCH, _F), jnp.float32),
            pltpu.SemaphoreType.DMA,
            pltpu.SemaphoreType.DMA,
            pltpu.SemaphoreType.DMA,
            pltpu.SemaphoreType.DMA,
            pltpu.SemaphoreType.DMA,
            pltpu.SemaphoreType.DMA,
        ],
    )
    def _combine(posa_hbm, posb_hbm, y_hbm, out_hbm, ia_v, ib_v,
                 ba0, ba1, bb0, bb1, sa0, sa1, sb0, sb1, sw0, sw1):
        wid = lax.axis_index("s") * 2 + lax.axis_index("c")
        pltpu.sync_copy(posa_hbm.at[wid], ia_v)
        pltpu.sync_copy(posb_hbm.at[wid], ib_v)
        base = wid * _TPT
        bas = (ba0, ba1)
        bbs = (bb0, bb1)
        sas = (sa0, sa1)
        sbs = (sb0, sb1)
        sws = (sw0, sw1)
        ga = [None, None]
        gb = [None, None]
        wrt = [None, None]

        def _do_chunk(c):
            q = c % 2
            ga[q].wait()
            gb[q].wait()
            ba, bb = bas[q], bbs[q]

            def row_body(r, carry):
                for d in range(_F // _LANES):
                    sl = pl.ds(d * _LANES, _LANES)
                    ba[r, sl] = ba[r, sl] + bb[r, sl]
                return carry

            lax.fori_loop(0, _CCH, row_body, 0)
            wrt[q] = pltpu.async_copy(
                ba, out_hbm.at[pl.ds(base + c * _CCH, _CCH)], sws[q])

        for c in range(_NCCH):
            p = c % 2
            if wrt[p] is not None:
                wrt[p].wait()
            ga[p] = pltpu.async_copy(y_hbm.at[ia_v.at[c]], bas[p], sas[p])
            gb[p] = pltpu.async_copy(y_hbm.at[ib_v.at[c]], bbs[p], sbs[p])
            if c >= 1:
                _do_chunk(c - 1)
        _do_chunk(_NCCH - 1)
        wrt[0].wait()
        wrt[1].wait()

    return _combine


def kernel(hidden_states, top_k_index, top_k_weights, w1, v1, w2):
    bsz = hidden_states.shape[0]
    hs = hidden_states.reshape(_S, _F)
    row_token, row_weight, pos_orig, meta = _route(top_k_index, top_k_weights)

    x = _gather_rows_kernel()(row_token.reshape(_NTILES, _NGCH, _GCH), hs)

    w1r = w1.reshape(_E, _F, _H)
    v1r = v1.reshape(_E, _F, _H)
    w2r = w2.reshape(_E, _F, _H)
    y = _expert_matmul(meta, x, w1r, v1r, w2r, row_weight.reshape(_NPAD, 1))

    pos2 = pos_orig.reshape(_S, _TOPK)
    posa = pos2[:, 0].reshape(_NTILES, _NCCH, _CCH)
    posb = pos2[:, 1].reshape(_NTILES, _NCCH, _CCH)
    out = _combine_kernel()(posa, posb, y)
    return out.reshape(bsz, _S, _F)
